# baseline (speedup 1.0000x reference)
import functools

import jax
import jax.numpy as jnp
from jax import lax
from jax.experimental import pallas as pl
from jax.experimental.pallas import tpu as pltpu
from jax.experimental.pallas import tpu_sc as plsc

_NUM_SEQS = 64
_VOCAB = 100000
_PROMPT_LEN = 512
_OUT_LEN = 128
_ROWS_PER_BLOCK = 8
_LANES = 16
_NUM_WORKERS = 32
_ROWS_PER_WORKER = _NUM_SEQS // _NUM_WORKERS
_PROMPT_DELTA = 65536


def _sc_bincount_body(ptoks, otoks, zeros, out, cnt_v, pt_v, ot_v):
    c = lax.axis_index("c")
    s = lax.axis_index("s")
    wid = s * 2 + c
    lane = lax.iota(jnp.int32, _LANES)
    pdelta = jnp.full((_LANES,), _PROMPT_DELTA, jnp.int32)
    odelta = jnp.full((_LANES,), 1, jnp.int32)
    zero16 = jnp.zeros((_LANES,), jnp.int32)
    pltpu.sync_copy(zeros, cnt_v)
    for j in range(_ROWS_PER_WORKER):
        r = wid * _ROWS_PER_WORKER + j
        pltpu.sync_copy(ptoks.at[r], pt_v)
        pltpu.sync_copy(otoks.at[r], ot_v)
        for g in range(_PROMPT_LEN // _LANES):
            idx = pt_v[pl.ds(g * _LANES, _LANES)]
            for ln in range(_LANES):
                plsc.addupdate_scatter(cnt_v, [idx], pdelta, mask=lane == ln)
        for g in range(_OUT_LEN // _LANES):
            idx = ot_v[pl.ds(g * _LANES, _LANES)]
            for ln in range(_LANES):
                plsc.addupdate_scatter(cnt_v, [idx], odelta, mask=lane == ln)
        pltpu.sync_copy(cnt_v, out.at[r])
        if j + 1 < _ROWS_PER_WORKER:
            for g in range(_PROMPT_LEN // _LANES):
                plsc.store_scatter(cnt_v, [pt_v[pl.ds(g * _LANES, _LANES)]],
                                   zero16)
            for g in range(_OUT_LEN // _LANES):
                plsc.store_scatter(cnt_v, [ot_v[pl.ds(g * _LANES, _LANES)]],
                                   zero16)


@functools.cache
def _sc_bincount():
    return pl.kernel(
        _sc_bincount_body,
        mesh=plsc.VectorSubcoreMesh(core_axis_name="c", subcore_axis_name="s"),
        compiler_params=pltpu.CompilerParams(needs_layout_passes=False),
        out_type=jax.ShapeDtypeStruct((_NUM_SEQS, _VOCAB), jnp.int32),
        scratch_types=[
            pltpu.VMEM((_VOCAB,), jnp.int32),
            pltpu.VMEM((_PROMPT_LEN,), jnp.int32),
            pltpu.VMEM((_OUT_LEN,), jnp.int32),
        ],
    )


_RCHUNK = 12544


def _rsum(x):
    n = x.shape[-1]
    parts = [jnp.sum(x[:, i:min(i + _RCHUNK, n)], axis=-1, keepdims=True)
             for i in range(0, n, _RCHUNK)]
    while len(parts) > 1:
        parts = [a + b for a, b in zip(parts[::2], parts[1::2])] + (
            [parts[-1]] if len(parts) % 2 else [])
    return parts[0]


def _rmax(x):
    n = x.shape[-1]
    parts = [jnp.max(x[:, i:min(i + _RCHUNK, n)], axis=-1, keepdims=True)
             for i in range(0, n, _RCHUNK)]
    while len(parts) > 1:
        parts = [jnp.maximum(a, b) for a, b in zip(parts[::2], parts[1::2])] + (
            [parts[-1]] if len(parts) % 2 else [])
    return parts[0]


def _dcount(mask, ones_f):
    x = jnp.where(mask, 1.0, 0.0)
    return lax.dot_general(x, ones_f, (((1,), (0,)), ((), ())),
                           preferred_element_type=jnp.float32)


def _dsum(x, ones_f):
    return lax.dot_general(x, ones_f, (((1,), (0,)), ((), ())),
                           preferred_element_type=jnp.float32)


def _bisect_trips(width):
    w31 = lax.bitcast_convert_type(width >> 1, jnp.int32)
    f = w31.astype(jnp.float32)
    expo = (lax.bitcast_convert_type(f, jnp.int32) >> 23) - 126
    bits = jnp.where(w31 > 0, expo + 1, 0)
    return jnp.max(bits)


def _tc_sampler_body(lg_ref, cnt_ref, rep_ref, frq_ref, prs_ref, tmp_ref,
                     topp_ref, minp_ref, topk_ref, out_ref):
    lg = lg_ref[...]
    cnt = cnt_ref[...]
    rep = rep_ref[...]
    frq = frq_ref[...]
    prs = prs_ref[...]
    tmp = tmp_ref[...]
    topp = topp_ref[...]
    minp = minp_ref[...]
    k = topk_ref[...]

    ones_f = jnp.ones((_VOCAB, 1), jnp.float32)

    ocnt = jnp.bitwise_and(cnt, 65535)
    omask = ocnt > 0
    touched = cnt > 0
    r = jnp.where(touched, rep, 1.0)
    z = jnp.where(lg > 0, lg / r, lg * r)
    z = z - frq * ocnt.astype(jnp.float32)
    z = z - prs * omask.astype(jnp.float32)
    t = jnp.where(tmp < 1e-2, 1.0, tmp)
    z = z / t

    m = _rmax(z)
    e = jnp.exp(z - m)

    u = lax.bitcast_convert_type(z, jnp.uint32)
    top = jnp.uint32(0x80000000)
    key = jnp.where(u >= top, ~u, u | top)
    key = jnp.where(z == 0.0, top, key)

    kk = jnp.clip(k, 1, _VOCAB)
    um = lax.bitcast_convert_type(m, jnp.uint32)
    mkey = jnp.where(um >= top, ~um, um | top)
    mkey = jnp.where(m == 0.0, top, mkey)

    radius = jnp.uint32(1 << 24)
    lo_guess = jnp.where(mkey >= radius, mkey - radius, jnp.uint32(0))
    kkf = kk.astype(jnp.float32)
    cnt_lo = _dcount(key >= lo_guess, ones_f)
    lo0 = jnp.where(cnt_lo >= kkf, lo_guess, jnp.uint32(0))

    def topk_bisect(_, c):
        lo, hi = c
        mid = lo + ((hi - lo) >> 1)
        cge = _dcount(key >= mid, ones_f)
        good = cge >= kkf
        return (jnp.where(good, mid, lo), jnp.where(good, hi, mid))

    hi0 = mkey + jnp.uint32(1)
    (K, _) = lax.fori_loop(0, _bisect_trips(hi0 - lo0), topk_bisect,
                           (lo0, hi0))
    ew = jnp.where(key >= K, e, 0.0)
    Z1 = _dsum(ew, ones_f)

    th = topp * Z1

    def topp_bisect(_, c):
        lo, hi = c
        mid = lo + ((hi - lo) >> 1)
        sg = _dsum(jnp.where(key > mid, ew, 0.0), ones_f)
        good = sg >= th
        return (jnp.where(good, mid, lo), jnp.where(good, hi, mid))

    lop = K - jnp.uint32(1)
    (Kp, _) = lax.fori_loop(0, _bisect_trips(hi0 - lop), topp_bisect,
                            (lop, hi0))

    kb = Kp + jnp.uint32(1)
    tie = key == kb
    tcnt = _dcount(tie, ones_f).astype(jnp.int32)
    col = lax.broadcasted_iota(jnp.int32, tie.shape, 1)

    def tie_correct(_):
        s_gt = _rsum(jnp.where(key > kb, ew, 0.0))
        q = _rmax(jnp.where(tie, ew, 0.0))
        qs = jnp.where(tcnt > 0, q, 1.0)
        n_keep = jnp.clip(jnp.ceil((th - s_gt) / qs), 0.0,
                          tcnt.astype(jnp.float32)).astype(jnp.int32)

        def tiecol_step(i, C):
            cand = C | (jnp.int32(1) << (16 - i))
            cnt = _rsum((tie & (col >= cand)).astype(jnp.int32))
            return jnp.where(cnt >= n_keep, cand, C)

        return lax.fori_loop(0, 17, tiecol_step,
                             jnp.zeros(kk.shape, jnp.int32))

    def tie_trivial(_):
        return jnp.zeros(kk.shape, jnp.int32)

    cstar = lax.cond(jnp.any(tcnt > 1), tie_correct, tie_trivial, 0)
    keep_tie = tie & (col >= cstar)

    w = jnp.where(((key > kb) | keep_tie) & (e >= minp), ew, 0.0)
    zs = _dsum(w, ones_f)
    out_ref[...] = w / zs


def _tc_sampler(logits, counts, rep, frq, prs, tmp, topp, minp, topk):
    n_blocks = _NUM_SEQS // _ROWS_PER_BLOCK
    row_spec = pl.BlockSpec((_ROWS_PER_BLOCK, _VOCAB), lambda i: (i, 0))
    par_spec = pl.BlockSpec((_ROWS_PER_BLOCK, 1), lambda i: (i, 0))
    return pl.pallas_call(
        _tc_sampler_body,
        grid=(n_blocks,),
        in_specs=[row_spec, row_spec] + [par_spec] * 7,
        out_specs=row_spec,
        out_shape=jax.ShapeDtypeStruct((_NUM_SEQS, _VOCAB), jnp.float32),
    )(logits, counts, rep, frq, prs, tmp, topp, minp, topk)


def kernel(logits, presence_penalties, frequency_penalties,
           repetition_penalties, temperatures, top_p, min_p, prompt_tokens,
           output_tokens, top_k):
    zeros = jnp.zeros((_VOCAB,), jnp.int32)
    counts = _sc_bincount()(prompt_tokens, output_tokens, zeros)
    col = lambda v: v.reshape(_NUM_SEQS, 1)
    return _tc_sampler(
        logits, counts, col(repetition_penalties), col(frequency_penalties),
        col(presence_penalties), col(temperatures), col(top_p), col(min_p),
        col(top_k))


# speedup vs baseline: 4.7044x; 4.7044x over previous
import functools

import jax
import jax.numpy as jnp
from jax import lax
from jax.experimental import pallas as pl
from jax.experimental.pallas import tpu as pltpu
from jax.experimental.pallas import tpu_sc as plsc

_NUM_SEQS = 64
_VOCAB = 100000
_PROMPT_LEN = 512
_OUT_LEN = 128
_ROWS_PER_BLOCK = 8
_LANES = 16
_NUM_WORKERS = 32
_ROWS_PER_WORKER = _NUM_SEQS // _NUM_WORKERS
_PROMPT_DELTA = 65536


def _sc_bincount_body(ptoks, otoks, zeros, out, cnt_v, pt_v, ot_v):
    c = lax.axis_index("c")
    s = lax.axis_index("s")
    wid = s * 2 + c
    lane = lax.iota(jnp.int32, _LANES)
    pdelta = jnp.full((_LANES,), _PROMPT_DELTA, jnp.int32)
    odelta = jnp.full((_LANES,), 1, jnp.int32)
    zero16 = jnp.zeros((_LANES,), jnp.int32)
    pltpu.sync_copy(zeros, cnt_v)
    for j in range(_ROWS_PER_WORKER):
        r = wid * _ROWS_PER_WORKER + j
        pltpu.sync_copy(ptoks.at[r], pt_v)
        pltpu.sync_copy(otoks.at[r], ot_v)
        for g in range(_PROMPT_LEN // _LANES):
            idx = pt_v[pl.ds(g * _LANES, _LANES)]
            for ln in range(_LANES):
                plsc.addupdate_scatter(cnt_v, [idx], pdelta, mask=lane == ln)
        for g in range(_OUT_LEN // _LANES):
            idx = ot_v[pl.ds(g * _LANES, _LANES)]
            for ln in range(_LANES):
                plsc.addupdate_scatter(cnt_v, [idx], odelta, mask=lane == ln)
        pltpu.sync_copy(cnt_v, out.at[r])
        if j + 1 < _ROWS_PER_WORKER:
            for g in range(_PROMPT_LEN // _LANES):
                plsc.store_scatter(cnt_v, [pt_v[pl.ds(g * _LANES, _LANES)]],
                                   zero16)
            for g in range(_OUT_LEN // _LANES):
                plsc.store_scatter(cnt_v, [ot_v[pl.ds(g * _LANES, _LANES)]],
                                   zero16)


@functools.cache
def _sc_bincount():
    return pl.kernel(
        _sc_bincount_body,
        mesh=plsc.VectorSubcoreMesh(core_axis_name="c", subcore_axis_name="s"),
        compiler_params=pltpu.CompilerParams(needs_layout_passes=False),
        out_type=jax.ShapeDtypeStruct((_NUM_SEQS, _VOCAB), jnp.int32),
        scratch_types=[
            pltpu.VMEM((_VOCAB,), jnp.int32),
            pltpu.VMEM((_PROMPT_LEN,), jnp.int32),
            pltpu.VMEM((_OUT_LEN,), jnp.int32),
        ],
    )


_RCHUNK = 12544


def _rsum(x):
    n = x.shape[-1]
    parts = [jnp.sum(x[:, i:min(i + _RCHUNK, n)], axis=-1, keepdims=True)
             for i in range(0, n, _RCHUNK)]
    while len(parts) > 1:
        parts = [a + b for a, b in zip(parts[::2], parts[1::2])] + (
            [parts[-1]] if len(parts) % 2 else [])
    return parts[0]


def _rmax(x):
    n = x.shape[-1]
    parts = [jnp.max(x[:, i:min(i + _RCHUNK, n)], axis=-1, keepdims=True)
             for i in range(0, n, _RCHUNK)]
    while len(parts) > 1:
        parts = [jnp.maximum(a, b) for a, b in zip(parts[::2], parts[1::2])] + (
            [parts[-1]] if len(parts) % 2 else [])
    return parts[0]


def _bisect_trips(width):
    w31 = lax.bitcast_convert_type(width >> 1, jnp.int32)
    f = w31.astype(jnp.float32)
    expo = (lax.bitcast_convert_type(f, jnp.int32) >> 23) - 126
    bits = jnp.where(w31 > 0, expo + 1, 0)
    return jnp.max(bits)


def _tc_sampler_body(lg_ref, cnt_ref, rep_ref, frq_ref, prs_ref, tmp_ref,
                     topp_ref, minp_ref, topk_ref, out_ref):
    lg = lg_ref[...]
    cnt = cnt_ref[...]
    rep = rep_ref[...]
    frq = frq_ref[...]
    prs = prs_ref[...]
    tmp = tmp_ref[...]
    topp = topp_ref[...]
    minp = minp_ref[...]
    k = topk_ref[...]

    ocnt = jnp.bitwise_and(cnt, 65535)
    omask = ocnt > 0
    touched = cnt > 0
    r = jnp.where(touched, rep, 1.0)
    z = jnp.where(lg > 0, lg / r, lg * r)
    z = z - frq * ocnt.astype(jnp.float32)
    z = z - prs * omask.astype(jnp.float32)
    t = jnp.where(tmp < 1e-2, 1.0, tmp)
    z = z / t

    m = _rmax(z)
    e = jnp.exp(z - m)

    u = lax.bitcast_convert_type(z, jnp.uint32)
    top = jnp.uint32(0x80000000)
    key = jnp.where(u >= top, ~u, u | top)
    key = jnp.where(z == 0.0, top, key)

    kk = jnp.clip(k, 1, _VOCAB)
    um = lax.bitcast_convert_type(m, jnp.uint32)
    mkey = jnp.where(um >= top, ~um, um | top)
    mkey = jnp.where(m == 0.0, top, mkey)

    radius = jnp.uint32(1 << 24)
    lo_guess = jnp.where(mkey >= radius, mkey - radius, jnp.uint32(0))
    kkf = kk.astype(jnp.float32)
    cnt_lo = _rsum((key >= lo_guess).astype(jnp.int32))
    lo0 = jnp.where(cnt_lo >= kk, lo_guess, jnp.uint32(0))

    def topk_bisect(_, c):
        lo, hi = c
        mid = lo + ((hi - lo) >> 1)
        cge = _rsum((key >= mid).astype(jnp.int32))
        good = cge >= kk
        return (jnp.where(good, mid, lo), jnp.where(good, hi, mid))

    hi0 = mkey + jnp.uint32(1)
    (K, _) = lax.fori_loop(0, _bisect_trips(hi0 - lo0), topk_bisect,
                           (lo0, hi0))
    ew = jnp.where(key >= K, e, 0.0)
    Z1 = _rsum(ew)

    th = topp * Z1

    def topp_bisect(_, c):
        lo, hi = c
        mid = lo + ((hi - lo) >> 1)
        sg = _rsum(jnp.where(key > mid, ew, 0.0))
        good = sg >= th
        return (jnp.where(good, mid, lo), jnp.where(good, hi, mid))

    lop = K - jnp.uint32(1)
    (Kp, _) = lax.fori_loop(0, _bisect_trips(hi0 - lop), topp_bisect,
                            (lop, hi0))

    kb = Kp + jnp.uint32(1)
    tie = key == kb
    tcnt = _rsum(tie.astype(jnp.int32))
    col = lax.broadcasted_iota(jnp.int32, tie.shape, 1)

    def tie_correct(_):
        s_gt = _rsum(jnp.where(key > kb, ew, 0.0))
        q = _rmax(jnp.where(tie, ew, 0.0))
        qs = jnp.where(tcnt > 0, q, 1.0)
        n_keep = jnp.clip(jnp.ceil((th - s_gt) / qs), 0.0,
                          tcnt.astype(jnp.float32)).astype(jnp.int32)

        def tiecol_step(i, C):
            cand = C | (jnp.int32(1) << (16 - i))
            cnt = _rsum((tie & (col >= cand)).astype(jnp.int32))
            return jnp.where(cnt >= n_keep, cand, C)

        return lax.fori_loop(0, 17, tiecol_step,
                             jnp.zeros(kk.shape, jnp.int32))

    def tie_trivial(_):
        return jnp.zeros(kk.shape, jnp.int32)

    cstar = lax.cond(jnp.any(tcnt > 1), tie_correct, tie_trivial, 0)
    keep_tie = tie & (col >= cstar)

    w = jnp.where(((key > kb) | keep_tie) & (e >= minp), ew, 0.0)
    zs = _rsum(w)
    out_ref[...] = w / zs


def _tc_sampler(logits, counts, rep, frq, prs, tmp, topp, minp, topk):
    n_blocks = _NUM_SEQS // _ROWS_PER_BLOCK
    row_spec = pl.BlockSpec((_ROWS_PER_BLOCK, _VOCAB), lambda i: (i, 0))
    par_spec = pl.BlockSpec((_ROWS_PER_BLOCK, 1), lambda i: (i, 0))
    return pl.pallas_call(
        _tc_sampler_body,
        grid=(n_blocks,),
        in_specs=[row_spec, row_spec] + [par_spec] * 7,
        out_specs=row_spec,
        out_shape=jax.ShapeDtypeStruct((_NUM_SEQS, _VOCAB), jnp.float32),
    )(logits, counts, rep, frq, prs, tmp, topp, minp, topk)


def kernel(logits, presence_penalties, frequency_penalties,
           repetition_penalties, temperatures, top_p, min_p, prompt_tokens,
           output_tokens, top_k):
    zeros = jnp.zeros((_VOCAB,), jnp.int32)
    counts = _sc_bincount()(prompt_tokens, output_tokens, zeros)
    col = lambda v: v.reshape(_NUM_SEQS, 1)
    return _tc_sampler(
        logits, counts, col(repetition_penalties), col(frequency_penalties),
        col(presence_penalties), col(temperatures), col(top_p), col(min_p),
        col(top_k))


# speedup vs baseline: 4.9570x; 1.0537x over previous
import functools

import jax
import jax.numpy as jnp
from jax import lax
from jax.experimental import pallas as pl
from jax.experimental.pallas import tpu as pltpu
from jax.experimental.pallas import tpu_sc as plsc

_NUM_SEQS = 64
_VOCAB = 100000
_PROMPT_LEN = 512
_OUT_LEN = 128
_ROWS_PER_BLOCK = 8
_LANES = 16
_NUM_WORKERS = 32
_ROWS_PER_WORKER = _NUM_SEQS // _NUM_WORKERS
_PROMPT_DELTA = 65536


def _sc_bincount_body(ptoks, otoks, zeros, out, cnt_v, pt_v, ot_v):
    c = lax.axis_index("c")
    s = lax.axis_index("s")
    wid = s * 2 + c
    lane = lax.iota(jnp.int32, _LANES)
    pdelta = jnp.full((_LANES,), _PROMPT_DELTA, jnp.int32)
    odelta = jnp.full((_LANES,), 1, jnp.int32)
    zero16 = jnp.zeros((_LANES,), jnp.int32)
    pltpu.sync_copy(zeros, cnt_v)
    for j in range(_ROWS_PER_WORKER):
        r = wid * _ROWS_PER_WORKER + j
        pltpu.sync_copy(ptoks.at[r], pt_v)
        pltpu.sync_copy(otoks.at[r], ot_v)
        for g in range(_PROMPT_LEN // _LANES):
            idx = pt_v[pl.ds(g * _LANES, _LANES)]
            for ln in range(_LANES):
                plsc.addupdate_scatter(cnt_v, [idx], pdelta, mask=lane == ln)
        for g in range(_OUT_LEN // _LANES):
            idx = ot_v[pl.ds(g * _LANES, _LANES)]
            for ln in range(_LANES):
                plsc.addupdate_scatter(cnt_v, [idx], odelta, mask=lane == ln)
        pltpu.sync_copy(cnt_v, out.at[r])
        if j + 1 < _ROWS_PER_WORKER:
            for g in range(_PROMPT_LEN // _LANES):
                plsc.store_scatter(cnt_v, [pt_v[pl.ds(g * _LANES, _LANES)]],
                                   zero16)
            for g in range(_OUT_LEN // _LANES):
                plsc.store_scatter(cnt_v, [ot_v[pl.ds(g * _LANES, _LANES)]],
                                   zero16)


@functools.cache
def _sc_bincount():
    return pl.kernel(
        _sc_bincount_body,
        mesh=plsc.VectorSubcoreMesh(core_axis_name="c", subcore_axis_name="s"),
        compiler_params=pltpu.CompilerParams(needs_layout_passes=False),
        out_type=jax.ShapeDtypeStruct((_NUM_SEQS, _VOCAB), jnp.int32),
        scratch_types=[
            pltpu.VMEM((_VOCAB,), jnp.int32),
            pltpu.VMEM((_PROMPT_LEN,), jnp.int32),
            pltpu.VMEM((_OUT_LEN,), jnp.int32),
        ],
    )


_RCHUNK = 6272


def _rsum(x):
    n = x.shape[-1]
    parts = [jnp.sum(x[:, i:min(i + _RCHUNK, n)], axis=-1, keepdims=True)
             for i in range(0, n, _RCHUNK)]
    while len(parts) > 1:
        parts = [a + b for a, b in zip(parts[::2], parts[1::2])] + (
            [parts[-1]] if len(parts) % 2 else [])
    return parts[0]


def _rmax(x):
    n = x.shape[-1]
    parts = [jnp.max(x[:, i:min(i + _RCHUNK, n)], axis=-1, keepdims=True)
             for i in range(0, n, _RCHUNK)]
    while len(parts) > 1:
        parts = [jnp.maximum(a, b) for a, b in zip(parts[::2], parts[1::2])] + (
            [parts[-1]] if len(parts) % 2 else [])
    return parts[0]


def _bisect_trips(width):
    w31 = lax.bitcast_convert_type(width >> 1, jnp.int32)
    f = w31.astype(jnp.float32)
    expo = (lax.bitcast_convert_type(f, jnp.int32) >> 23) - 126
    bits = jnp.where(w31 > 0, expo + 1, 0)
    return jnp.max(bits)


def _tc_sampler_body(lg_ref, cnt_ref, rep_ref, frq_ref, prs_ref, tmp_ref,
                     topp_ref, minp_ref, topk_ref, out_ref):
    lg = lg_ref[...]
    cnt = cnt_ref[...]
    rep = rep_ref[...]
    frq = frq_ref[...]
    prs = prs_ref[...]
    tmp = tmp_ref[...]
    topp = topp_ref[...]
    minp = minp_ref[...]
    k = topk_ref[...]

    ocnt = jnp.bitwise_and(cnt, 65535)
    omask = ocnt > 0
    touched = cnt > 0
    r = jnp.where(touched, rep, 1.0)
    z = jnp.where(lg > 0, lg / r, lg * r)
    z = z - frq * ocnt.astype(jnp.float32)
    z = z - prs * omask.astype(jnp.float32)
    t = jnp.where(tmp < 1e-2, 1.0, tmp)
    z = z / t

    m = _rmax(z)
    e = jnp.exp(z - m)

    u = lax.bitcast_convert_type(z, jnp.uint32)
    top = jnp.uint32(0x80000000)
    key = jnp.where(u >= top, ~u, u | top)
    key = jnp.where(z == 0.0, top, key)

    kk = jnp.clip(k, 1, _VOCAB)
    um = lax.bitcast_convert_type(m, jnp.uint32)
    mkey = jnp.where(um >= top, ~um, um | top)
    mkey = jnp.where(m == 0.0, top, mkey)

    radius = jnp.uint32(1 << 24)
    lo_guess = jnp.where(mkey >= radius, mkey - radius, jnp.uint32(0))
    kkf = kk.astype(jnp.float32)
    cnt_lo = _rsum((key >= lo_guess).astype(jnp.int32))
    lo0 = jnp.where(cnt_lo >= kk, lo_guess, jnp.uint32(0))

    def topk_bisect(_, c):
        lo, hi = c
        mid = lo + ((hi - lo) >> 1)
        cge = _rsum((key >= mid).astype(jnp.int32))
        good = cge >= kk
        return (jnp.where(good, mid, lo), jnp.where(good, hi, mid))

    hi0 = mkey + jnp.uint32(1)
    (K, _) = lax.fori_loop(0, _bisect_trips(hi0 - lo0), topk_bisect,
                           (lo0, hi0))
    ew = jnp.where(key >= K, e, 0.0)
    Z1 = _rsum(ew)

    th = topp * Z1

    def topp_bisect(_, c):
        lo, hi = c
        mid = lo + ((hi - lo) >> 1)
        sg = _rsum(jnp.where(key > mid, ew, 0.0))
        good = sg >= th
        return (jnp.where(good, mid, lo), jnp.where(good, hi, mid))

    lop = K - jnp.uint32(1)
    (Kp, _) = lax.fori_loop(0, _bisect_trips(hi0 - lop), topp_bisect,
                            (lop, hi0))

    kb = Kp + jnp.uint32(1)
    tie = key == kb
    tcnt = _rsum(tie.astype(jnp.int32))
    col = lax.broadcasted_iota(jnp.int32, tie.shape, 1)

    def tie_correct(_):
        s_gt = _rsum(jnp.where(key > kb, ew, 0.0))
        q = _rmax(jnp.where(tie, ew, 0.0))
        qs = jnp.where(tcnt > 0, q, 1.0)
        n_keep = jnp.clip(jnp.ceil((th - s_gt) / qs), 0.0,
                          tcnt.astype(jnp.float32)).astype(jnp.int32)

        def tiecol_step(i, C):
            cand = C | (jnp.int32(1) << (16 - i))
            cnt = _rsum((tie & (col >= cand)).astype(jnp.int32))
            return jnp.where(cnt >= n_keep, cand, C)

        return lax.fori_loop(0, 17, tiecol_step,
                             jnp.zeros(kk.shape, jnp.int32))

    def tie_trivial(_):
        return jnp.zeros(kk.shape, jnp.int32)

    cstar = lax.cond(jnp.any(tcnt > 1), tie_correct, tie_trivial, 0)
    keep_tie = tie & (col >= cstar)

    w = jnp.where(((key > kb) | keep_tie) & (e >= minp), ew, 0.0)
    zs = _rsum(w)
    out_ref[...] = w / zs


def _tc_sampler(logits, counts, rep, frq, prs, tmp, topp, minp, topk):
    n_blocks = _NUM_SEQS // _ROWS_PER_BLOCK
    row_spec = pl.BlockSpec((_ROWS_PER_BLOCK, _VOCAB), lambda i: (i, 0))
    par_spec = pl.BlockSpec((_ROWS_PER_BLOCK, 1), lambda i: (i, 0))
    return pl.pallas_call(
        _tc_sampler_body,
        grid=(n_blocks,),
        in_specs=[row_spec, row_spec] + [par_spec] * 7,
        out_specs=row_spec,
        out_shape=jax.ShapeDtypeStruct((_NUM_SEQS, _VOCAB), jnp.float32),
    )(logits, counts, rep, frq, prs, tmp, topp, minp, topk)


def kernel(logits, presence_penalties, frequency_penalties,
           repetition_penalties, temperatures, top_p, min_p, prompt_tokens,
           output_tokens, top_k):
    zeros = jnp.zeros((_VOCAB,), jnp.int32)
    counts = _sc_bincount()(prompt_tokens, output_tokens, zeros)
    col = lambda v: v.reshape(_NUM_SEQS, 1)
    return _tc_sampler(
        logits, counts, col(repetition_penalties), col(frequency_penalties),
        col(presence_penalties), col(temperatures), col(top_p), col(min_p),
        col(top_k))


# speedup vs baseline: 5.0071x; 1.0101x over previous
import functools

import jax
import jax.numpy as jnp
from jax import lax
from jax.experimental import pallas as pl
from jax.experimental.pallas import tpu as pltpu
from jax.experimental.pallas import tpu_sc as plsc

_NUM_SEQS = 64
_VOCAB = 100000
_PROMPT_LEN = 512
_OUT_LEN = 128
_ROWS_PER_BLOCK = 8
_LANES = 16
_NUM_WORKERS = 32
_ROWS_PER_WORKER = _NUM_SEQS // _NUM_WORKERS
_PROMPT_DELTA = 65536


def _sc_bincount_body(ptoks, otoks, zeros, out, cnt_v, pt_v, ot_v):
    c = lax.axis_index("c")
    s = lax.axis_index("s")
    wid = s * 2 + c
    lane = lax.iota(jnp.int32, _LANES)
    pdelta = jnp.full((_LANES,), _PROMPT_DELTA, jnp.int32)
    odelta = jnp.full((_LANES,), 1, jnp.int32)
    zero16 = jnp.zeros((_LANES,), jnp.int32)
    pltpu.sync_copy(zeros, cnt_v)
    for j in range(_ROWS_PER_WORKER):
        r = wid * _ROWS_PER_WORKER + j
        pltpu.sync_copy(ptoks.at[r], pt_v)
        pltpu.sync_copy(otoks.at[r], ot_v)
        for g in range(_PROMPT_LEN // _LANES):
            idx = pt_v[pl.ds(g * _LANES, _LANES)]
            for ln in range(_LANES):
                plsc.addupdate_scatter(cnt_v, [idx], pdelta, mask=lane == ln)
        for g in range(_OUT_LEN // _LANES):
            idx = ot_v[pl.ds(g * _LANES, _LANES)]
            for ln in range(_LANES):
                plsc.addupdate_scatter(cnt_v, [idx], odelta, mask=lane == ln)
        pltpu.sync_copy(cnt_v, out.at[r])
        if j + 1 < _ROWS_PER_WORKER:
            for g in range(_PROMPT_LEN // _LANES):
                plsc.store_scatter(cnt_v, [pt_v[pl.ds(g * _LANES, _LANES)]],
                                   zero16)
            for g in range(_OUT_LEN // _LANES):
                plsc.store_scatter(cnt_v, [ot_v[pl.ds(g * _LANES, _LANES)]],
                                   zero16)


@functools.cache
def _sc_bincount():
    return pl.kernel(
        _sc_bincount_body,
        mesh=plsc.VectorSubcoreMesh(core_axis_name="c", subcore_axis_name="s"),
        compiler_params=pltpu.CompilerParams(needs_layout_passes=False),
        out_type=jax.ShapeDtypeStruct((_NUM_SEQS, _VOCAB), jnp.int32),
        scratch_types=[
            pltpu.VMEM((_VOCAB,), jnp.int32),
            pltpu.VMEM((_PROMPT_LEN,), jnp.int32),
            pltpu.VMEM((_OUT_LEN,), jnp.int32),
        ],
    )


_RCHUNK = 3200


def _rsum(x):
    n = x.shape[-1]
    parts = [jnp.sum(x[:, i:min(i + _RCHUNK, n)], axis=-1, keepdims=True)
             for i in range(0, n, _RCHUNK)]
    while len(parts) > 1:
        parts = [a + b for a, b in zip(parts[::2], parts[1::2])] + (
            [parts[-1]] if len(parts) % 2 else [])
    return parts[0]


def _rmax(x):
    n = x.shape[-1]
    parts = [jnp.max(x[:, i:min(i + _RCHUNK, n)], axis=-1, keepdims=True)
             for i in range(0, n, _RCHUNK)]
    while len(parts) > 1:
        parts = [jnp.maximum(a, b) for a, b in zip(parts[::2], parts[1::2])] + (
            [parts[-1]] if len(parts) % 2 else [])
    return parts[0]


def _bisect_trips(width):
    w31 = lax.bitcast_convert_type(width >> 1, jnp.int32)
    f = w31.astype(jnp.float32)
    expo = (lax.bitcast_convert_type(f, jnp.int32) >> 23) - 126
    bits = jnp.where(w31 > 0, expo + 1, 0)
    return jnp.max(bits)


def _tc_sampler_body(lg_ref, cnt_ref, rep_ref, frq_ref, prs_ref, tmp_ref,
                     topp_ref, minp_ref, topk_ref, out_ref):
    lg = lg_ref[...]
    cnt = cnt_ref[...]
    rep = rep_ref[...]
    frq = frq_ref[...]
    prs = prs_ref[...]
    tmp = tmp_ref[...]
    topp = topp_ref[...]
    minp = minp_ref[...]
    k = topk_ref[...]

    ocnt = jnp.bitwise_and(cnt, 65535)
    omask = ocnt > 0
    touched = cnt > 0
    r = jnp.where(touched, rep, 1.0)
    z = jnp.where(lg > 0, lg / r, lg * r)
    z = z - frq * ocnt.astype(jnp.float32)
    z = z - prs * omask.astype(jnp.float32)
    t = jnp.where(tmp < 1e-2, 1.0, tmp)
    z = z / t

    m = _rmax(z)
    e = jnp.exp(z - m)

    u = lax.bitcast_convert_type(z, jnp.uint32)
    top = jnp.uint32(0x80000000)
    key = jnp.where(u >= top, ~u, u | top)
    key = jnp.where(z == 0.0, top, key)

    kk = jnp.clip(k, 1, _VOCAB)
    um = lax.bitcast_convert_type(m, jnp.uint32)
    mkey = jnp.where(um >= top, ~um, um | top)
    mkey = jnp.where(m == 0.0, top, mkey)

    radius = jnp.uint32(1 << 24)
    lo_guess = jnp.where(mkey >= radius, mkey - radius, jnp.uint32(0))
    kkf = kk.astype(jnp.float32)
    cnt_lo = _rsum((key >= lo_guess).astype(jnp.int32))
    lo0 = jnp.where(cnt_lo >= kk, lo_guess, jnp.uint32(0))

    def topk_bisect(_, c):
        lo, hi = c
        mid = lo + ((hi - lo) >> 1)
        cge = _rsum((key >= mid).astype(jnp.int32))
        good = cge >= kk
        return (jnp.where(good, mid, lo), jnp.where(good, hi, mid))

    hi0 = mkey + jnp.uint32(1)
    (K, _) = lax.fori_loop(0, _bisect_trips(hi0 - lo0), topk_bisect,
                           (lo0, hi0))
    ew = jnp.where(key >= K, e, 0.0)
    Z1 = _rsum(ew)

    th = topp * Z1

    def topp_bisect(_, c):
        lo, hi = c
        mid = lo + ((hi - lo) >> 1)
        sg = _rsum(jnp.where(key > mid, ew, 0.0))
        good = sg >= th
        return (jnp.where(good, mid, lo), jnp.where(good, hi, mid))

    lop = K - jnp.uint32(1)
    (Kp, _) = lax.fori_loop(0, _bisect_trips(hi0 - lop), topp_bisect,
                            (lop, hi0))

    kb = Kp + jnp.uint32(1)
    tie = key == kb
    tcnt = _rsum(tie.astype(jnp.int32))
    col = lax.broadcasted_iota(jnp.int32, tie.shape, 1)

    def tie_correct(_):
        s_gt = _rsum(jnp.where(key > kb, ew, 0.0))
        q = _rmax(jnp.where(tie, ew, 0.0))
        qs = jnp.where(tcnt > 0, q, 1.0)
        n_keep = jnp.clip(jnp.ceil((th - s_gt) / qs), 0.0,
                          tcnt.astype(jnp.float32)).astype(jnp.int32)

        def tiecol_step(i, C):
            cand = C | (jnp.int32(1) << (16 - i))
            cnt = _rsum((tie & (col >= cand)).astype(jnp.int32))
            return jnp.where(cnt >= n_keep, cand, C)

        return lax.fori_loop(0, 17, tiecol_step,
                             jnp.zeros(kk.shape, jnp.int32))

    def tie_trivial(_):
        return jnp.zeros(kk.shape, jnp.int32)

    cstar = lax.cond(jnp.any(tcnt > 1), tie_correct, tie_trivial, 0)
    keep_tie = tie & (col >= cstar)

    w = jnp.where(((key > kb) | keep_tie) & (e >= minp), ew, 0.0)
    zs = _rsum(w)
    out_ref[...] = w / zs


def _tc_sampler(logits, counts, rep, frq, prs, tmp, topp, minp, topk):
    n_blocks = _NUM_SEQS // _ROWS_PER_BLOCK
    row_spec = pl.BlockSpec((_ROWS_PER_BLOCK, _VOCAB), lambda i: (i, 0))
    par_spec = pl.BlockSpec((_ROWS_PER_BLOCK, 1), lambda i: (i, 0))
    return pl.pallas_call(
        _tc_sampler_body,
        grid=(n_blocks,),
        in_specs=[row_spec, row_spec] + [par_spec] * 7,
        out_specs=row_spec,
        out_shape=jax.ShapeDtypeStruct((_NUM_SEQS, _VOCAB), jnp.float32),
    )(logits, counts, rep, frq, prs, tmp, topp, minp, topk)


def kernel(logits, presence_penalties, frequency_penalties,
           repetition_penalties, temperatures, top_p, min_p, prompt_tokens,
           output_tokens, top_k):
    zeros = jnp.zeros((_VOCAB,), jnp.int32)
    counts = _sc_bincount()(prompt_tokens, output_tokens, zeros)
    col = lambda v: v.reshape(_NUM_SEQS, 1)
    return _tc_sampler(
        logits, counts, col(repetition_penalties), col(frequency_penalties),
        col(presence_penalties), col(temperatures), col(top_p), col(min_p),
        col(top_k))
